# baseline (speedup 1.0000x reference)
import jax
import jax.numpy as jnp
from jax import lax
from jax.experimental import pallas as pl
from jax.experimental.pallas import tpu as pltpu
from jax.experimental.pallas import tpu_sc as plsc

NUM_EMB = 1000000
DIM = 32
BATCH = 16384
HIST = 50
NC = 2
NS = 16

CHUNK = 2048
CPH = BATCH // CHUNK
NCH = HIST * CPH
CPT = NCH // NS
IDX_RES = CPT * CHUNK
DPC = DIM // NC
SSEG = 62496
SREM = NUM_EMB - NS * SSEG


def _emb_body(idxT, wT, outT, idx_all, row_sh, gbuf, sem):
    cid = lax.axis_index("c")
    sid = lax.axis_index("s")

    def load_idx(j, carry):
        c = sid * CPT + j
        h = c // CPH
        b0 = (c % CPH) * CHUNK
        pltpu.sync_copy(idxT.at[h, pl.ds(b0, CHUNK)],
                        idx_all.at[pl.ds(j * CHUNK, CHUNK)])
        return carry

    lax.fori_loop(0, CPT, load_idx, 0)

    def per_d(dd, carry):
        d = cid * DPC + dd
        plsc.subcore_barrier()
        pltpu.sync_copy(wT.at[d, pl.ds(sid * SSEG, SSEG)],
                        row_sh.at[pl.ds(sid * SSEG, SSEG)])

        @pl.when(sid == 0)
        def _():
            pltpu.sync_copy(wT.at[d, pl.ds(NS * SSEG, SREM)],
                            row_sh.at[pl.ds(NS * SSEG, SREM)])

        plsc.subcore_barrier()

        def per_chunk(j, inner):
            c = sid * CPT + j
            h = c // CPH
            b0 = (c % CPH) * CHUNK
            pltpu.async_copy(row_sh.at[idx_all.at[pl.ds(j * CHUNK, CHUNK)]],
                             gbuf, sem).wait()
            pltpu.sync_copy(gbuf, outT.at[h, d, pl.ds(b0, CHUNK)])
            return inner

        lax.fori_loop(0, CPT, per_chunk, 0)
        return carry

    lax.fori_loop(0, DPC, per_d, 0)


def _embed(idxT, wT):
    mesh = plsc.VectorSubcoreMesh(core_axis_name="c", subcore_axis_name="s")
    return pl.kernel(
        _emb_body,
        mesh=mesh,
        out_type=jax.ShapeDtypeStruct((HIST, DIM, BATCH), jnp.float32),
        scratch_types=[
            pltpu.VMEM((IDX_RES,), jnp.int32),
            pltpu.VMEM_SHARED((NUM_EMB,), jnp.float32),
            pltpu.VMEM((CHUNK,), jnp.float32),
            pltpu.SemaphoreType.DMA,
        ],
        compiler_params=pltpu.CompilerParams(use_tc_tiling_on_sc=False),
    )(idxT, wT)


def kernel(input_, weight):
    idxT = input_.T.astype(jnp.int32)
    wT = weight.T
    outT = _embed(idxT, wT)
    return jnp.transpose(outT, (2, 0, 1))
